# baseline (speedup 1.0000x reference)
import functools

import jax
import jax.numpy as jnp
from jax import lax
from jax.experimental import pallas as pl
from jax.experimental.pallas import tpu as pltpu
from jax.experimental.pallas import tpu_sc as plsc

_L = 16
_N = 32768
_ROWS = 64
_NW = 32
_ROWS_PER_W = _ROWS // _NW
_NA = 65280
_NB = 32768
_NC = 256
_U = 4


def _lanes():
    return lax.iota(jnp.int32, _L)


def _gather16(x, idx):
    dn = lax.GatherDimensionNumbers(
        offset_dims=(), collapsed_slice_dims=(0,), start_index_map=(0,))
    return lax.gather(
        x, idx.reshape(_L, 1), dn, slice_sizes=(1,),
        mode=lax.GatherScatterMode.PROMISE_IN_BOUNDS)


def _cumsum16(x):
    lanes = _lanes()
    s = x
    for d in (1, 2, 4, 8):
        sh = _gather16(s, jnp.maximum(lanes - d, 0))
        s = s + jnp.where(lanes >= d, sh, 0)
    return s


def _top_lane():
    return jnp.full((_L,), _L - 1, jnp.int32)


def _scan(load16, j, ngroups):
    zero = jnp.zeros((_L,), jnp.int32)

    def body(i, carry):
        above, d_star, j_next, done = carry
        g = ngroups - 1 - i
        h = load16(g)
        cs = _cumsum16(h)
        gsum = _gather16(cs, _top_lane())
        s_vec = above + gsum - cs + h
        m = s_vec >= j
        pc = plsc.all_reduce_population_count(m)
        hit = jnp.logical_and(pc > 0, done == 0)
        lanepos = jnp.maximum(pc - 1, 0)
        cs_at = _gather16(cs, lanepos)
        d_star = jnp.where(hit, g * _L + lanepos, d_star)
        j_next = jnp.where(hit, j - (above + gsum - cs_at), j_next)
        done = jnp.where(pc > 0, 1, done)
        above = jnp.where(done > 0, above, above + gsum)
        return above, d_star, j_next, done

    _, d_star, j_next, _ = lax.fori_loop(0, ngroups, body, (zero, zero, j, zero))
    return d_star, j_next


def _scan_slice(cnt_ref, j, ngroups):
    return _scan(lambda g: cnt_ref[pl.ds(g * _L, _L)], j, ngroups)


def _scan_gather(hist_ref, base, j, ngroups):
    lanes = _lanes()
    return _scan(
        lambda g: plsc.load_gather(hist_ref, [base + g * _L + lanes]),
        j, ngroups)


def _zero_range(ref, nwords):
    z = jnp.zeros((_L,), jnp.int32)

    @plsc.parallel_loop(0, nwords, _L, unroll=_U)
    def _(i):
        ref[pl.ds(i, _L)] = z


def _pass_a(row_ref, ha_ref, hb_ref):
    ones = jnp.ones((_L,), jnp.int32)

    @plsc.parallel_loop(0, _N, _L, unroll=_U)
    def _(i):
        v = row_ref[pl.ds(i, _L)]
        b = plsc.bitcast(v, jnp.int32) & jnp.int32(0x7FFFFFFF)
        plsc.addupdate_scatter(ha_ref, [lax.shift_right_logical(b, 15)], ones)
        plsc.addupdate_scatter(hb_ref, [lax.shift_right_logical(b, 23)], ones)


def _pass_b(row_ref, ha_ref, hb_ref, d_a):
    ones = jnp.ones((_L,), jnp.int32)

    @plsc.parallel_loop(0, _N, _L, unroll=_U)
    def _(i):
        v = row_ref[pl.ds(i, _L)]
        b = plsc.bitcast(v, jnp.int32) & jnp.int32(0x7FFFFFFF)
        m = lax.shift_right_logical(b, 15) == d_a
        dig = b & jnp.int32(0x7FFF)
        plsc.addupdate_scatter(hb_ref, [dig], ones, mask=m)
        plsc.addupdate_scatter(
            ha_ref, [lax.shift_right_logical(dig, 7)], ones, mask=m)


def _apply_pass(row_ref, hb_ref, thr):
    z = jnp.zeros((_L,), jnp.int32)

    @plsc.parallel_loop(0, _N, _L, unroll=_U)
    def _(i):
        v = row_ref[pl.ds(i, _L)]
        b = plsc.bitcast(v, jnp.int32) & jnp.int32(0x7FFFFFFF)
        row_ref[pl.ds(i, _L)] = jnp.where(b >= thr, v, jnp.float32(0.0))
        plsc.store_scatter(hb_ref, [b & jnp.int32(0x7FFF)], z)


_MESH = plsc.VectorSubcoreMesh(core_axis_name="c", subcore_axis_name="s")


@functools.partial(
    pl.kernel,
    mesh=_MESH,
    compiler_params=pltpu.CompilerParams(needs_layout_passes=False),
    out_type=jax.ShapeDtypeStruct((_ROWS, _N), jnp.float32),
    scratch_types=[
        pltpu.VMEM((_N,), jnp.float32),
        pltpu.VMEM((_NA,), jnp.int32),
        pltpu.VMEM((_NB,), jnp.int32),
        pltpu.VMEM((_L,), jnp.int32),
        pltpu.SemaphoreType.DMA,
        pltpu.SemaphoreType.DMA,
    ],
)
def _sc_topk(scores_hbm, kvec_hbm, zeros_hbm, out_hbm,
             row_v, ha_v, hb_v, k_v, sem, zsem):
    wid = lax.axis_index("s") * 2 + lax.axis_index("c")
    row0 = wid * _ROWS_PER_W
    in0 = pltpu.async_copy(scores_hbm.at[row0], row_v, sem)
    pltpu.sync_copy(kvec_hbm, k_v)
    kk = k_v[...]
    _zero_range(ha_v, _NA)
    _zero_range(hb_v, _NB)
    in0.wait()
    zdma = None
    for r in range(_ROWS_PER_W):
        row = row0 + r
        if r > 0:
            pltpu.sync_copy(scores_hbm.at[row], row_v)
        _pass_a(row_v, ha_v, hb_v)
        c_a, j_a = _scan_slice(hb_v, kk, 16)
        dra, j_b = _scan_gather(ha_v, c_a * 256, j_a, 16)
        d_a = c_a * 256 + dra
        zdma = pltpu.async_copy(zeros_hbm, ha_v.at[pl.ds(_NC, _NA - _NC)], zsem)
        _zero_range(hb_v, _NC)
        _zero_range(ha_v, _NC)
        _pass_b(row_v, ha_v, hb_v, d_a)
        c_b, j_c = _scan_slice(ha_v, j_b, 16)
        drb, _ = _scan_gather(hb_v, c_b * 128, j_c, 8)
        d_b = c_b * 128 + drb
        _zero_range(ha_v, _NC)
        thr = d_a * jnp.int32(32768) + d_b
        _apply_pass(row_v, hb_v, thr)
        pltpu.sync_copy(row_v, out_hbm.at[row])
        zdma.wait()


def kernel(scores, k):
    kvec = jnp.full((_L,), k, jnp.int32)
    zeros = jnp.zeros((_NA - _NC,), jnp.int32)
    return _sc_topk(scores, kvec, zeros)


# speedup vs baseline: 1.0386x; 1.0386x over previous
import functools

import jax
import jax.numpy as jnp
from jax import lax
from jax.experimental import pallas as pl
from jax.experimental.pallas import tpu as pltpu
from jax.experimental.pallas import tpu_sc as plsc

_L = 16
_N = 32768
_ROWS = 64
_NW = 32
_ROWS_PER_W = _ROWS // _NW
_NA = 65280
_NB = 32768
_NC = 256
_U = 8


def _lanes():
    return lax.iota(jnp.int32, _L)


def _gather16(x, idx):
    dn = lax.GatherDimensionNumbers(
        offset_dims=(), collapsed_slice_dims=(0,), start_index_map=(0,))
    return lax.gather(
        x, idx.reshape(_L, 1), dn, slice_sizes=(1,),
        mode=lax.GatherScatterMode.PROMISE_IN_BOUNDS)


def _cumsum16(x):
    lanes = _lanes()
    s = x
    for d in (1, 2, 4, 8):
        sh = _gather16(s, jnp.maximum(lanes - d, 0))
        s = s + jnp.where(lanes >= d, sh, 0)
    return s


def _top_lane():
    return jnp.full((_L,), _L - 1, jnp.int32)


def _scan(load16, j, ngroups):
    zero = jnp.zeros((_L,), jnp.int32)

    def body(i, carry):
        above, d_star, j_next, done = carry
        g = ngroups - 1 - i
        h = load16(g)
        cs = _cumsum16(h)
        gsum = _gather16(cs, _top_lane())
        s_vec = above + gsum - cs + h
        m = s_vec >= j
        pc = plsc.all_reduce_population_count(m)
        hit = jnp.logical_and(pc > 0, done == 0)
        lanepos = jnp.maximum(pc - 1, 0)
        cs_at = _gather16(cs, lanepos)
        d_star = jnp.where(hit, g * _L + lanepos, d_star)
        j_next = jnp.where(hit, j - (above + gsum - cs_at), j_next)
        done = jnp.where(pc > 0, 1, done)
        above = jnp.where(done > 0, above, above + gsum)
        return above, d_star, j_next, done

    _, d_star, j_next, _ = lax.fori_loop(0, ngroups, body, (zero, zero, j, zero))
    return d_star, j_next


def _scan_slice(cnt_ref, j, ngroups):
    return _scan(lambda g: cnt_ref[pl.ds(g * _L, _L)], j, ngroups)


def _scan_gather(hist_ref, base, j, ngroups):
    lanes = _lanes()
    return _scan(
        lambda g: plsc.load_gather(hist_ref, [base + g * _L + lanes]),
        j, ngroups)


def _zero_range(ref, nwords):
    z = jnp.zeros((_L,), jnp.int32)

    @plsc.parallel_loop(0, nwords, _L, unroll=_U)
    def _(i):
        ref[pl.ds(i, _L)] = z


def _pass_a(row_ref, ha_ref, hb_ref):
    ones = jnp.ones((_L,), jnp.int32)

    @plsc.parallel_loop(0, _N, _L, unroll=_U)
    def _(i):
        v = row_ref[pl.ds(i, _L)]
        b = plsc.bitcast(v, jnp.int32) & jnp.int32(0x7FFFFFFF)
        plsc.addupdate_scatter(ha_ref, [lax.shift_right_logical(b, 15)], ones)
        plsc.addupdate_scatter(hb_ref, [lax.shift_right_logical(b, 23)], ones)


def _pass_b(row_ref, ha_ref, hb_ref, d_a):
    ones = jnp.ones((_L,), jnp.int32)

    @plsc.parallel_loop(0, _N, _L, unroll=_U)
    def _(i):
        v = row_ref[pl.ds(i, _L)]
        b = plsc.bitcast(v, jnp.int32) & jnp.int32(0x7FFFFFFF)
        m = lax.shift_right_logical(b, 15) == d_a
        dig = b & jnp.int32(0x7FFF)
        plsc.addupdate_scatter(hb_ref, [dig], ones, mask=m)
        plsc.addupdate_scatter(
            ha_ref, [lax.shift_right_logical(dig, 7)], ones, mask=m)


def _apply_pass(row_ref, hb_ref, thr):
    z = jnp.zeros((_L,), jnp.int32)

    @plsc.parallel_loop(0, _N, _L, unroll=_U)
    def _(i):
        v = row_ref[pl.ds(i, _L)]
        b = plsc.bitcast(v, jnp.int32) & jnp.int32(0x7FFFFFFF)
        row_ref[pl.ds(i, _L)] = jnp.where(b >= thr, v, jnp.float32(0.0))
        plsc.store_scatter(hb_ref, [b & jnp.int32(0x7FFF)], z)


_MESH = plsc.VectorSubcoreMesh(core_axis_name="c", subcore_axis_name="s")


@functools.partial(
    pl.kernel,
    mesh=_MESH,
    compiler_params=pltpu.CompilerParams(needs_layout_passes=False),
    out_type=jax.ShapeDtypeStruct((_ROWS, _N), jnp.float32),
    scratch_types=[
        pltpu.VMEM((_N,), jnp.float32),
        pltpu.VMEM((_NA,), jnp.int32),
        pltpu.VMEM((_NB,), jnp.int32),
        pltpu.VMEM((_L,), jnp.int32),
        pltpu.SemaphoreType.DMA,
        pltpu.SemaphoreType.DMA,
    ],
)
def _sc_topk(scores_hbm, kvec_hbm, zeros_hbm, out_hbm,
             row_v, ha_v, hb_v, k_v, sem, zsem):
    wid = lax.axis_index("s") * 2 + lax.axis_index("c")
    row0 = wid * _ROWS_PER_W
    in0 = pltpu.async_copy(scores_hbm.at[row0], row_v, sem)
    pltpu.sync_copy(kvec_hbm, k_v)
    kk = k_v[...]
    _zero_range(ha_v, _NA)
    _zero_range(hb_v, _NB)
    in0.wait()
    zdma = None
    for r in range(_ROWS_PER_W):
        row = row0 + r
        if r > 0:
            pltpu.sync_copy(scores_hbm.at[row], row_v)
        _pass_a(row_v, ha_v, hb_v)
        c_a, j_a = _scan_slice(hb_v, kk, 16)
        dra, j_b = _scan_gather(ha_v, c_a * 256, j_a, 16)
        d_a = c_a * 256 + dra
        zdma = pltpu.async_copy(zeros_hbm, ha_v.at[pl.ds(_NC, _NA - _NC)], zsem)
        _zero_range(hb_v, _NC)
        _zero_range(ha_v, _NC)
        _pass_b(row_v, ha_v, hb_v, d_a)
        c_b, j_c = _scan_slice(ha_v, j_b, 16)
        drb, _ = _scan_gather(hb_v, c_b * 128, j_c, 8)
        d_b = c_b * 128 + drb
        _zero_range(ha_v, _NC)
        thr = d_a * jnp.int32(32768) + d_b
        _apply_pass(row_v, hb_v, thr)
        pltpu.sync_copy(row_v, out_hbm.at[row])
        zdma.wait()


def kernel(scores, k):
    kvec = jnp.full((_L,), k, jnp.int32)
    zeros = jnp.zeros((_NA - _NC,), jnp.int32)
    return _sc_topk(scores, kvec, zeros)
